# baseline (speedup 1.0000x reference)
import functools

import jax
import jax.numpy as jnp
from jax import lax
from jax.experimental import pallas as pl
from jax.experimental.pallas import tpu as pltpu
from jax.experimental.pallas import tpu_sc as plsc

_INFO = plsc.get_sparse_core_info()
_NC = _INFO.num_cores
_NS = _INFO.num_subcores
_NW = _NC * _NS

_CHUNK = 128


def _embedding_gather(idx3, weight, n, d):
    nw, n_chunks, chunk = idx3.shape
    b_per_w = n_chunks * chunk
    mesh = plsc.VectorSubcoreMesh(core_axis_name="c", subcore_axis_name="s")

    @functools.partial(
        pl.kernel,
        out_type=jax.ShapeDtypeStruct((n, d), jnp.float32),
        mesh=mesh,
        scratch_types=[
            pltpu.VMEM((n_chunks, chunk), jnp.int32),
            pltpu.VMEM((chunk, d), jnp.float32),
            pltpu.SemaphoreType.DMA,
        ],
        compiler_params=pltpu.CompilerParams(use_tc_tiling_on_sc=False),
    )
    def k(idx_hbm, table_hbm, out_hbm, idx_v, rows_v, gsem):
        wid = lax.axis_index("s") * _NC + lax.axis_index("c")
        base = wid * b_per_w
        pltpu.sync_copy(idx_hbm.at[wid], idx_v)

        def body(j, carry):
            pltpu.async_copy(table_hbm.at[idx_v.at[j]], rows_v, gsem).wait()
            pltpu.sync_copy(rows_v, out_hbm.at[pl.ds(base + j * chunk, chunk)])
            return carry

        lax.fori_loop(0, n_chunks, body, 0)

    return k(idx3, weight)


def kernel(input_, weight):
    b, h = input_.shape
    n = b * h
    d = weight.shape[1]
    idx = input_.reshape(-1).astype(jnp.int32)
    b_per_w = n // _NW
    idx3 = idx.reshape(_NW, b_per_w // _CHUNK, _CHUNK)
    out = _embedding_gather(idx3, weight, n, d)
    return out.reshape(b, h, d)


# speedup vs baseline: 1.0594x; 1.0594x over previous
import functools

import jax
import jax.numpy as jnp
from jax import lax
from jax.experimental import pallas as pl
from jax.experimental.pallas import tpu as pltpu
from jax.experimental.pallas import tpu_sc as plsc

_INFO = plsc.get_sparse_core_info()
_NC = _INFO.num_cores
_NS = _INFO.num_subcores
_NW = _NC * _NS

_R = 32


def _embedding_gather(idx, weight):
    b, h = idx.shape
    v, d = weight.shape
    rows_per_w = b // _NW
    n_chunks = rows_per_w // _R
    mesh = plsc.VectorSubcoreMesh(core_axis_name="c", subcore_axis_name="s")

    @functools.partial(
        pl.kernel,
        out_type=jax.ShapeDtypeStruct((b, h, d), jnp.float32),
        mesh=mesh,
        scratch_types=[
            pltpu.VMEM((rows_per_w, h), jnp.int32),
            pltpu.VMEM((2, _R, h, d), jnp.float32),
            pltpu.SemaphoreType.DMA,
            pltpu.SemaphoreType.DMA,
        ],
        compiler_params=pltpu.CompilerParams(use_tc_tiling_on_sc=False),
    )
    def k(idx_hbm, table_hbm, out_hbm, idx_v, rows_v, gsem, osem):
        wid = lax.axis_index("s") * _NC + lax.axis_index("c")
        base = wid * rows_per_w
        pltpu.sync_copy(idx_hbm.at[pl.ds(base, rows_per_w)], idx_v)

        def gd(j, rr, bf):
            src = table_hbm.at[idx_v.at[j * _R + rr]]
            return pltpu.make_async_copy(src, rows_v.at[bf, rr], gsem)

        def od(j, bf):
            dst = out_hbm.at[pl.ds(base + j * _R, _R)]
            return pltpu.make_async_copy(rows_v.at[bf], dst, osem)

        for rr in range(_R):
            gd(0, rr, 0).start()

        def body(j, carry):
            bf = lax.rem(j, 2)
            for rr in range(_R):
                gd(j, rr, bf).wait()

            @pl.when(j >= 1)
            def _():
                od(j - 1, 1 - bf).wait()

            @pl.when(j + 1 < n_chunks)
            def _():
                for rr in range(_R):
                    gd(j + 1, rr, 1 - bf).start()

            od(j, bf).start()
            return carry

        lax.fori_loop(0, n_chunks, body, 0)
        od(n_chunks - 1, (n_chunks - 1) % 2).wait()

    return k(idx, weight)


def kernel(input_, weight):
    return _embedding_gather(input_.astype(jnp.int32), weight)


# speedup vs baseline: 1.5139x; 1.4291x over previous
import functools

import jax
import jax.numpy as jnp
from jax import lax
from jax.experimental import pallas as pl
from jax.experimental.pallas import tpu as pltpu
from jax.experimental.pallas import tpu_sc as plsc

_INFO = plsc.get_sparse_core_info()
_NC = _INFO.num_cores
_NS = _INFO.num_subcores
_NW = _NC * _NS

_R = 8
_IDX_STAGE = 128


def _embedding_gather(idx, weight):
    b, h = idx.shape
    v, d = weight.shape
    rows_per_w = b // _NW
    n_chunks = rows_per_w // _R
    mesh = plsc.VectorSubcoreMesh(core_axis_name="c", subcore_axis_name="s")

    @functools.partial(
        pl.kernel,
        out_type=jax.ShapeDtypeStruct((b, h, d), jnp.float32),
        mesh=mesh,
        scratch_types=[
            pltpu.VMEM((_IDX_STAGE, h), jnp.int32),
            pltpu.VMEM((2, _R, h, d), jnp.float32),
            pltpu.SemaphoreType.DMA,
            pltpu.SemaphoreType.DMA,
        ],
    )
    def k(idx_hbm, table_hbm, out_hbm, idx_v, rows_v, gsem, osem):
        wid = lax.axis_index("s") * _NC + lax.axis_index("c")
        base = wid * rows_per_w
        chunks_per_stage = _IDX_STAGE // _R

        def stage_idx(stage):
            pltpu.sync_copy(
                idx_hbm.at[pl.ds(base + stage * _IDX_STAGE, _IDX_STAGE)], idx_v
            )

        def fire_chunk(j, bf):
            for rr in range(_R):
                r = lax.rem(j, chunks_per_stage) * _R + rr
                va = idx_v[r, pl.ds(0, 16)]
                vb = idx_v[r, pl.ds(h - 16, 16)]
                for c in range(h):
                    i = va[c] if c < 16 else vb[c - (h - 16)]
                    pltpu.make_async_copy(
                        table_hbm.at[pl.ds(i, 1)],
                        rows_v.at[bf, rr, pl.ds(c, 1)],
                        gsem,
                    ).start()

        def chunk_wait(j, bf):
            pltpu.make_async_copy(
                out_hbm.at[pl.ds(base, _R)], rows_v.at[bf], gsem
            ).wait()

        def od(j, bf):
            dst = out_hbm.at[pl.ds(base + j * _R, _R)]
            return pltpu.make_async_copy(rows_v.at[bf], dst, osem)

        stage_idx(0)
        fire_chunk(0, 0)

        def body(j, carry):
            bf = lax.rem(j, 2)
            chunk_wait(j, bf)

            @pl.when(j >= 1)
            def _():
                od(j - 1, 1 - bf).wait()

            @pl.when(lax.rem(j + 1, chunks_per_stage) == 0)
            def _():
                @pl.when(j + 1 < n_chunks)
                def _():
                    stage_idx((j + 1) // chunks_per_stage)

            @pl.when(j + 1 < n_chunks)
            def _():
                fire_chunk(j + 1, 1 - bf)

            od(j, bf).start()
            return carry

        lax.fori_loop(0, n_chunks, body, 0)
        od(n_chunks - 1, (n_chunks - 1) % 2).wait()

    return k(idx, weight)


def kernel(input_, weight):
    return _embedding_gather(input_.astype(jnp.int32), weight)
